# Initial kernel scaffold; baseline (speedup 1.0000x reference)
import jax
import jax.numpy as jnp
from jax.experimental import pallas as pl


def kernel(x, edge_index, node_label_index, node_label, W0, b0, W1, b1, W2, b2, W3, b3, Wp, bp):
    raise NotImplementedError("write your pallas kernel here")


# speedup vs baseline: 9.1648x; 9.1648x over previous
import functools

import jax
import jax.numpy as jnp
from jax import lax
from jax.experimental import pallas as pl
from jax.experimental.pallas import tpu as pltpu
import jax.experimental.pallas.tpu_sc as plsc

_N = 10000
_E = 320000
_D = 128
_C = 40

_NC = 2
_NS = 16
_NW = _NC * _NS
_K = 128
_NCHUNK = 80
_EPW = _NCHUNK * _K
_EP = _NW * _EPW
_NP = 10112
_RPS = _NP // _NS

_MESH = plsc.VectorSubcoreMesh(
    core_axis_name="c", subcore_axis_name="s",
    num_cores=_NC, num_subcores=_NS)


def _wid():
    return lax.axis_index("s") * _NC + lax.axis_index("c")


def _deg_body(eidx_hbm, ones_hbm, zer_hbm, deg_hbm, dtab, eidx, ones):
    cid = lax.axis_index("c")
    sid = lax.axis_index("s")
    wid = _wid()
    r0 = pl.multiple_of(sid * _RPS, 8)
    pltpu.sync_copy(zer_hbm.at[pl.ds(r0, _RPS)], dtab.at[pl.ds(r0, _RPS)])
    pltpu.sync_copy(ones_hbm, ones)
    pltpu.sync_copy(eidx_hbm.at[wid], eidx)
    plsc.subcore_barrier()

    def step(c, carry):
        pltpu.sync_copy(ones, dtab.at[eidx.at[c, 1]], add=True)
        return carry

    lax.fori_loop(0, _NCHUNK, step, 0)
    plsc.subcore_barrier()
    pltpu.sync_copy(dtab.at[pl.ds(r0, _RPS)], deg_hbm.at[cid, pl.ds(r0, _RPS)])


_deg_kernel = functools.partial(
    pl.kernel,
    out_type=jax.ShapeDtypeStruct((_NC, _NP, 16), jnp.float32),
    mesh=_MESH,
    scratch_types=[
        pltpu.VMEM_SHARED((_NP, 16), jnp.float32),
        pltpu.VMEM((_NCHUNK, 2, _K), jnp.int32),
        pltpu.VMEM((_K, 16), jnp.float32),
    ],
)(_deg_body)


def _adj_body(u_hbm, eidx_hbm, zer_hbm, y_hbm, acc, ibuf, rows, sem_i, sem_r):
    cid = lax.axis_index("c")
    sid = lax.axis_index("s")
    wid = _wid()
    r0 = pl.multiple_of(sid * _RPS, 8)
    pltpu.sync_copy(zer_hbm.at[pl.ds(r0, _RPS)], acc.at[pl.ds(r0, _RPS)])

    def idx_copy(c):
        s = lax.rem(c, 3)
        return pltpu.make_async_copy(
            eidx_hbm.at[wid, c], ibuf.at[s], sem_i.at[s])

    def gat_copy(c):
        return pltpu.make_async_copy(
            u_hbm.at[ibuf.at[lax.rem(c, 3), 0]],
            rows.at[lax.rem(c, 2)], sem_r.at[lax.rem(c, 2)])

    plsc.subcore_barrier()

    idx_copy(0).start()
    idx_copy(1).start()
    idx_copy(0).wait()
    gat_copy(0).start()

    def step(c, carry):
        gat_copy(c).wait()

        @pl.when(c + 2 < _NCHUNK)
        def _():
            idx_copy(c + 2).start()

        @pl.when(c + 1 < _NCHUNK)
        def _():
            idx_copy(c + 1).wait()
            gat_copy(c + 1).start()

        pltpu.sync_copy(rows.at[lax.rem(c, 2)],
                        acc.at[ibuf.at[lax.rem(c, 3), 1]], add=True)
        return carry

    lax.fori_loop(0, _NCHUNK, step, 0)
    plsc.subcore_barrier()
    pltpu.sync_copy(acc.at[pl.ds(r0, _RPS)], y_hbm.at[cid, pl.ds(r0, _RPS)])


_adj_kernel = functools.partial(
    pl.kernel,
    out_type=jax.ShapeDtypeStruct((_NC, _NP, _D), jnp.float32),
    mesh=_MESH,
    scratch_types=[
        pltpu.VMEM_SHARED((_NP, _D), jnp.float32),
        pltpu.VMEM((3, 2, _K), jnp.int32),
        pltpu.VMEM((2, _K, _D), jnp.float32),
        pltpu.SemaphoreType.DMA((3,)),
        pltpu.SemaphoreType.DMA((2,)),
    ],
)(_adj_body)


def _pre_body(degp_ref, x_ref, dinv_ref, ux_ref):
    deg = degp_ref[0, :_N, 0:1] + degp_ref[1, :_N, 0:1] + 1.0
    dinv = lax.rsqrt(jnp.maximum(deg, 1e-12))
    dinv_ref[...] = dinv
    ux_ref[...] = x_ref[...] * dinv


def _stage_body(yp_ref, u_ref, dinv_ref, gin_ref, W_ref, b_ref,
                u_out_ref, g_out_ref, nb: float):
    dinv = dinv_ref[...]
    g = dinv * (yp_ref[0, :_N, :] + yp_ref[1, :_N, :] + u_ref[...])
    gsum = g + gin_ref[...]
    cell = jnp.dot(gsum, W_ref[...],
                   preferred_element_type=jnp.float32) + nb * b_ref[...]
    u_out_ref[...] = dinv * jnp.maximum(cell, 0.0)
    g_out_ref[...] = gsum


def _final_body(yp_ref, u_ref, dinv_ref, gin_ref, W_ref, b_ref,
                Wp_ref, bp_ref, pred_ref):
    dinv = dinv_ref[...]
    g = dinv * (yp_ref[0, :_N, :] + yp_ref[1, :_N, :] + u_ref[...])
    gsum = g + gin_ref[...]
    cell = jnp.dot(gsum, W_ref[...],
                   preferred_element_type=jnp.float32) + 3.0 * b_ref[...]
    m = jnp.max(cell, axis=1, keepdims=True)
    ex = jnp.exp(cell - m)
    lse = jnp.log(jnp.sum(ex, axis=1, keepdims=True))
    feat = cell - m - lse
    pred_ref[...] = jnp.dot(feat, Wp_ref[...],
                            preferred_element_type=jnp.float32) + bp_ref[...]


def _tc(body, out_shape, *args):
    return pl.pallas_call(body, out_shape=out_shape)(*args)


def kernel(x, edge_index, node_label_index, node_label,
           W0, b0, W1, b1, W2, b2, W3, b3, Wp, bp):
    f32 = jnp.float32
    pad = _EP - _E
    src = jnp.concatenate(
        [edge_index[0], jnp.zeros((pad,), jnp.int32)]).reshape(_NW, _NCHUNK, _K)
    dst = jnp.concatenate(
        [edge_index[1], jnp.full((pad,), _N, jnp.int32)]).reshape(_NW, _NCHUNK, _K)
    eidx = jnp.stack([src, dst], axis=2)
    zer16 = jnp.zeros((_NP, 16), f32)
    ones16 = jnp.ones((_K, 16), f32)
    zerD = jnp.zeros((_NP, _D), f32)
    gzero = jnp.zeros((_N, _D), f32)

    degp = _deg_kernel(eidx, ones16, zer16)

    dinv, ux = _tc(_pre_body,
                   (jax.ShapeDtypeStruct((_N, 1), f32),
                    jax.ShapeDtypeStruct((_N, _D), f32)),
                   degp, x)

    shp_u = jax.ShapeDtypeStruct((_N, _D), f32)
    stage1 = functools.partial(_stage_body, nb=1.0)
    stage2 = functools.partial(_stage_body, nb=2.0)

    yx = _adj_kernel(ux, eidx, zerD)
    u0, _g = _tc(stage1, (shp_u, shp_u), yx, ux, dinv, gzero,
                 W0, b0.reshape(1, _D))
    y0 = _adj_kernel(u0, eidx, zerD)
    u1, g0 = _tc(stage1, (shp_u, shp_u), y0, u0, dinv, gzero,
                 W1, b1.reshape(1, _D))
    y1 = _adj_kernel(u1, eidx, zerD)
    u2, g01 = _tc(stage2, (shp_u, shp_u), y1, u1, dinv, g0,
                  W2, b2.reshape(1, _D))
    y2 = _adj_kernel(u2, eidx, zerD)
    pred = _tc(_final_body, jax.ShapeDtypeStruct((_N, _C), f32),
               y2, u2, dinv, g01, W3, b3.reshape(1, _D),
               Wp, bp.reshape(1, _C))

    pred = jnp.take(pred, node_label_index, axis=0)
    return pred, node_label


# speedup vs baseline: 9.6785x; 1.0561x over previous
import functools

import jax
import jax.numpy as jnp
from jax import lax
from jax.experimental import pallas as pl
from jax.experimental.pallas import tpu as pltpu
import jax.experimental.pallas.tpu_sc as plsc

_N = 10000
_E = 320000
_D = 128
_C = 40

_NC = 2
_NS = 16
_NW = _NC * _NS
_K = 128
_NCHUNK = 80
_TOTCH = _NW * _NCHUNK
_C0 = 120
_C1 = 2 * _NCHUNK - _C0
_CMAX = max(_C0, _C1)
_EP = _TOTCH * _K
_NP = 10112
_RPS = _NP // _NS

_MESH = plsc.VectorSubcoreMesh(
    core_axis_name="c", subcore_axis_name="s",
    num_cores=_NC, num_subcores=_NS)


def _wid():
    return lax.axis_index("s") * _NC + lax.axis_index("c")


def _core_range():
    cid = lax.axis_index("c")
    sid = lax.axis_index("s")
    nch = _C0 + cid * (_C1 - _C0)
    base = cid * (_NS * _C0) + sid * nch
    return base, nch


def _deg_body(eidx_hbm, ones_hbm, zer_hbm, deg_hbm, dtab, eidx, ones):
    cid = lax.axis_index("c")
    sid = lax.axis_index("s")
    base, nch = _core_range()
    r0 = pl.multiple_of(sid * _RPS, 8)
    pltpu.sync_copy(zer_hbm.at[pl.ds(r0, _RPS)], dtab.at[pl.ds(r0, _RPS)])
    pltpu.sync_copy(ones_hbm, ones)
    plsc.subcore_barrier()

    def step(c, carry):
        @pl.when(c < nch)
        def _():
            pltpu.sync_copy(eidx_hbm.at[base + c], eidx)
            pltpu.sync_copy(ones, dtab.at[eidx.at[1]], add=True)

        return carry

    lax.fori_loop(0, _CMAX, step, 0)
    plsc.subcore_barrier()
    pltpu.sync_copy(dtab.at[pl.ds(r0, _RPS)], deg_hbm.at[cid, pl.ds(r0, _RPS)])


_deg_kernel = functools.partial(
    pl.kernel,
    out_type=jax.ShapeDtypeStruct((_NC, _NP, 16), jnp.float32),
    mesh=_MESH,
    scratch_types=[
        pltpu.VMEM_SHARED((_NP, 16), jnp.float32),
        pltpu.VMEM((2, _K), jnp.int32),
        pltpu.VMEM((_K, 16), jnp.float32),
    ],
)(_deg_body)


def _adj_body(u_hbm, eidx_hbm, zer_hbm, y_hbm, acc, ibuf, rows, sem_i, sem_r):
    cid = lax.axis_index("c")
    sid = lax.axis_index("s")
    base, nch = _core_range()
    r0 = pl.multiple_of(sid * _RPS, 8)
    pltpu.sync_copy(zer_hbm.at[pl.ds(r0, _RPS)], acc.at[pl.ds(r0, _RPS)])

    def idx_copy(c):
        s = lax.rem(c, 3)
        return pltpu.make_async_copy(
            eidx_hbm.at[base + c], ibuf.at[s], sem_i.at[s])

    def gat_copy(c):
        return pltpu.make_async_copy(
            u_hbm.at[ibuf.at[lax.rem(c, 3), 0]],
            rows.at[lax.rem(c, 2)], sem_r.at[lax.rem(c, 2)])

    plsc.subcore_barrier()

    idx_copy(0).start()
    idx_copy(1).start()
    idx_copy(0).wait()
    gat_copy(0).start()

    def step(c, carry):
        @pl.when(c < nch)
        def _():
            gat_copy(c).wait()

        @pl.when(c + 2 < nch)
        def _():
            idx_copy(c + 2).start()

        @pl.when(c + 1 < nch)
        def _():
            idx_copy(c + 1).wait()
            gat_copy(c + 1).start()

        @pl.when(c < nch)
        def _():
            pltpu.sync_copy(rows.at[lax.rem(c, 2)],
                            acc.at[ibuf.at[lax.rem(c, 3), 1]], add=True)

        return carry

    lax.fori_loop(0, _CMAX, step, 0)
    plsc.subcore_barrier()
    pltpu.sync_copy(acc.at[pl.ds(r0, _RPS)], y_hbm.at[cid, pl.ds(r0, _RPS)])


_adj_kernel = functools.partial(
    pl.kernel,
    out_type=jax.ShapeDtypeStruct((_NC, _NP, _D), jnp.float32),
    mesh=_MESH,
    scratch_types=[
        pltpu.VMEM_SHARED((_NP, _D), jnp.float32),
        pltpu.VMEM((3, 2, _K), jnp.int32),
        pltpu.VMEM((2, _K, _D), jnp.float32),
        pltpu.SemaphoreType.DMA((3,)),
        pltpu.SemaphoreType.DMA((2,)),
    ],
)(_adj_body)


def _pre_body(degp_ref, x_ref, dinv_ref, ux_ref):
    deg = degp_ref[0, :_N, 0:1] + degp_ref[1, :_N, 0:1] + 1.0
    dinv = lax.rsqrt(jnp.maximum(deg, 1e-12))
    dinv_ref[...] = dinv
    ux_ref[...] = x_ref[...] * dinv


def _stage_body(yp_ref, u_ref, dinv_ref, gin_ref, W_ref, b_ref,
                u_out_ref, g_out_ref, nb: float):
    dinv = dinv_ref[...]
    g = dinv * (yp_ref[0, :_N, :] + yp_ref[1, :_N, :] + u_ref[...])
    gsum = g + gin_ref[...]
    cell = jnp.dot(gsum, W_ref[...],
                   preferred_element_type=jnp.float32) + nb * b_ref[...]
    u_out_ref[...] = dinv * jnp.maximum(cell, 0.0)
    g_out_ref[...] = gsum


def _final_body(yp_ref, u_ref, dinv_ref, gin_ref, W_ref, b_ref,
                Wp_ref, bp_ref, pred_ref):
    dinv = dinv_ref[...]
    g = dinv * (yp_ref[0, :_N, :] + yp_ref[1, :_N, :] + u_ref[...])
    gsum = g + gin_ref[...]
    cell = jnp.dot(gsum, W_ref[...],
                   preferred_element_type=jnp.float32) + 3.0 * b_ref[...]
    m = jnp.max(cell, axis=1, keepdims=True)
    ex = jnp.exp(cell - m)
    lse = jnp.log(jnp.sum(ex, axis=1, keepdims=True))
    feat = cell - m - lse
    pred_ref[...] = jnp.dot(feat, Wp_ref[...],
                            preferred_element_type=jnp.float32) + bp_ref[...]


def _tc(body, out_shape, *args):
    return pl.pallas_call(body, out_shape=out_shape)(*args)


def kernel(x, edge_index, node_label_index, node_label,
           W0, b0, W1, b1, W2, b2, W3, b3, Wp, bp):
    f32 = jnp.float32
    pad = _EP - _E
    src = jnp.concatenate(
        [edge_index[0], jnp.zeros((pad,), jnp.int32)]).reshape(_TOTCH, _K)
    dst = jnp.concatenate(
        [edge_index[1], jnp.full((pad,), _N, jnp.int32)]).reshape(_TOTCH, _K)
    eidx = jnp.stack([src, dst], axis=1)
    zer16 = jnp.zeros((_NP, 16), f32)
    ones16 = jnp.ones((_K, 16), f32)
    zerD = jnp.zeros((_NP, _D), f32)
    gzero = jnp.zeros((_N, _D), f32)

    degp = _deg_kernel(eidx, ones16, zer16)

    dinv, ux = _tc(_pre_body,
                   (jax.ShapeDtypeStruct((_N, 1), f32),
                    jax.ShapeDtypeStruct((_N, _D), f32)),
                   degp, x)

    shp_u = jax.ShapeDtypeStruct((_N, _D), f32)
    stage1 = functools.partial(_stage_body, nb=1.0)
    stage2 = functools.partial(_stage_body, nb=2.0)

    yx = _adj_kernel(ux, eidx, zerD)
    u0, _g = _tc(stage1, (shp_u, shp_u), yx, ux, dinv, gzero,
                 W0, b0.reshape(1, _D))
    y0 = _adj_kernel(u0, eidx, zerD)
    u1, g0 = _tc(stage1, (shp_u, shp_u), y0, u0, dinv, gzero,
                 W1, b1.reshape(1, _D))
    y1 = _adj_kernel(u1, eidx, zerD)
    u2, g01 = _tc(stage2, (shp_u, shp_u), y1, u1, dinv, g0,
                  W2, b2.reshape(1, _D))
    y2 = _adj_kernel(u2, eidx, zerD)
    pred = _tc(_final_body, jax.ShapeDtypeStruct((_N, _C), f32),
               y2, u2, dinv, g01, W3, b3.reshape(1, _D),
               Wp, bp.reshape(1, _C))

    pred = jnp.take(pred, node_label_index, axis=0)
    return pred, node_label
